# baseline (speedup 1.0000x reference)
import functools

import jax
import jax.numpy as jnp
from jax import lax
from jax.experimental import pallas as pl
from jax.experimental.pallas import tpu as pltpu
from jax.experimental.pallas import tpu_sc as plsc

_NUM_EXPERT = 8
_IN = 1024
_OUT = 1024
_TOKENS = 2048
_BT = 512
_NCHUNK = 2
_CROWS = _TOKENS // _NCHUNK
_CNB = _CROWS // _BT
_CNW = _CNB + _NUM_EXPERT - 1


def _sc_mesh():
    return plsc.VectorSubcoreMesh(core_axis_name="c", subcore_axis_name="s")


def _sc_inverse(dest):
    t = dest.shape[0]
    mesh = _sc_mesh()
    nworker = mesh.num_cores * mesh.num_subcores
    per_w = t // nworker

    @functools.partial(
        pl.kernel,
        mesh=mesh,
        out_type=jax.ShapeDtypeStruct((t,), jnp.int32),
        scratch_types=[
            pltpu.VMEM((per_w,), jnp.int32),
            pltpu.VMEM((per_w,), jnp.int32),
            pltpu.SemaphoreType.DMA,
        ],
    )
    def k(dest_hbm, out_hbm, idx_v, val_v, sem):
        wid = lax.axis_index("s") * mesh.num_cores + lax.axis_index("c")
        base = wid * per_w
        pltpu.sync_copy(dest_hbm.at[pl.ds(base, per_w)], idx_v)
        for j in range(per_w // 16):
            val_v[pl.ds(j * 16, 16)] = (
                base + j * 16 + lax.iota(jnp.int32, 16))
        pltpu.async_copy(val_v, out_hbm.at[idx_v], sem).wait()

    return k(dest)


def _sc_gather_chunk(src, sidx, chunk):
    feat = src.shape[1]
    mesh = _sc_mesh()
    nworker = mesh.num_cores * mesh.num_subcores
    per_w = _CROWS // nworker

    @functools.partial(
        pl.kernel,
        mesh=mesh,
        out_type=jax.ShapeDtypeStruct((_CROWS, feat), src.dtype),
        scratch_types=[
            pltpu.VMEM((per_w,), jnp.int32),
            pltpu.VMEM((per_w, feat), src.dtype),
            pltpu.SemaphoreType.DMA,
        ],
    )
    def k(src_hbm, sidx_hbm, out_hbm, idx_v, rows_v, sem):
        wid = lax.axis_index("s") * mesh.num_cores + lax.axis_index("c")
        base = wid * per_w
        pltpu.sync_copy(sidx_hbm.at[pl.ds(chunk * _CROWS + base, per_w)],
                        idx_v)
        pltpu.async_copy(src_hbm.at[idx_v], rows_v, sem).wait()
        pltpu.sync_copy(rows_v, out_hbm.at[pl.ds(base, per_w)])

    return k(src, sidx)


def _sc_scatter_rows(ya, yb, sidx):
    half, feat = ya.shape
    t = sidx.shape[0]
    mesh = _sc_mesh()
    nworker = mesh.num_cores * mesh.num_subcores
    per_w = t // nworker
    w_split = half // per_w

    @functools.partial(
        pl.kernel,
        mesh=mesh,
        out_type=jax.ShapeDtypeStruct((t, feat), ya.dtype),
        scratch_types=[
            pltpu.VMEM((per_w,), jnp.int32),
            pltpu.VMEM((per_w, feat), ya.dtype),
            pltpu.SemaphoreType.DMA,
        ],
    )
    def k(ya_hbm, yb_hbm, idx_hbm, out_hbm, idx_v, rows_v, sem):
        wid = lax.axis_index("s") * mesh.num_cores + lax.axis_index("c")
        base = wid * per_w
        pltpu.sync_copy(idx_hbm.at[pl.ds(base, per_w)], idx_v)

        @pl.when(wid < w_split)
        def _():
            pltpu.sync_copy(ya_hbm.at[pl.ds(base, per_w)], rows_v)

        @pl.when(wid >= w_split)
        def _():
            pltpu.sync_copy(yb_hbm.at[pl.ds(base - half, per_w)], rows_v)

        pltpu.async_copy(rows_v, out_hbm.at[idx_v], sem).wait()

    return k(ya, yb, sidx)


def _mm_body(meta_ref, x_ref, w_ref, o_ref):
    w = pl.program_id(0)
    first = meta_ref[2, w]
    lo = meta_ref[3, w]
    hi = meta_ref[4, w]
    rows = meta_ref[0, w] * _BT + lax.broadcasted_iota(
        jnp.int32, (_BT, 1), 0)
    mask = (rows >= lo) & (rows < hi)
    xm = jnp.where(mask, x_ref[...], 0.0)
    part = lax.dot_general(
        xm, w_ref[0],
        dimension_numbers=(((1,), (1,)), ((), ())),
        preferred_element_type=jnp.float32,
    )

    @pl.when(first == 1)
    def _():
        o_ref[...] = part

    @pl.when(first == 0)
    def _():
        o_ref[...] += part


def _grouped_matmul(x_sorted, weight, meta):
    rows = x_sorted.shape[0]
    grid_spec = pltpu.PrefetchScalarGridSpec(
        num_scalar_prefetch=1,
        grid=(meta.shape[1],),
        in_specs=[
            pl.BlockSpec((_BT, _IN), lambda w, m: (m[0, w], 0)),
            pl.BlockSpec((1, _OUT, _IN), lambda w, m: (m[1, w], 0, 0)),
        ],
        out_specs=pl.BlockSpec((_BT, _OUT), lambda w, m: (m[0, w], 0)),
    )
    return pl.pallas_call(
        _mm_body,
        grid_spec=grid_spec,
        out_shape=jax.ShapeDtypeStruct((rows, _OUT), jnp.float32),
        compiler_params=pltpu.CompilerParams(
            dimension_semantics=("arbitrary",),
        ),
    )(meta, x_sorted, weight)


def _routing(gate):
    g = gate.astype(jnp.int32)
    t = g.shape[0]
    eids = jnp.arange(_NUM_EXPERT, dtype=jnp.int32)
    oh3 = (g.reshape(128, t // 128, 1) == eids).astype(jnp.int32)
    c1 = jnp.cumsum(oh3, axis=1)
    row_tot = c1[:, -1, :]
    c2 = jnp.cumsum(row_tot, axis=0)
    excl = (c1 - oh3) + (c2 - row_tot)[:, None, :]
    pos = jnp.sum(oh3 * excl, axis=2).reshape(t)
    counts = c2[-1]
    off_end = jnp.cumsum(counts)
    off = off_end - counts
    seg = jnp.sum(oh3 * off[None, None, :], axis=2).reshape(t)
    dest = (pos + seg).astype(jnp.int32)
    metas = []
    warr = jnp.arange(_CNW, dtype=jnp.int32)
    for c in range(_NCHUNK):
        blk_lo = (c * _CROWS
                  + jnp.arange(_CNB, dtype=jnp.int32) * _BT)
        e_lo = jnp.sum((off[None, :] <= blk_lo[:, None]).astype(jnp.int32),
                       axis=1) - 1
        e_hi = jnp.sum((off[None, :] <= blk_lo[:, None] + (_BT - 1)
                        ).astype(jnp.int32), axis=1) - 1
        nitem = e_hi - e_lo + 1
        starts = jnp.cumsum(nitem) - nitem
        total = jnp.sum(nitem)
        b_of = jnp.sum((warr[:, None] >= starts[None, :]).astype(jnp.int32),
                       axis=1) - 1
        e_w = e_lo[b_of] + warr - starts[b_of]
        valid = warr < total
        e_load = jnp.clip(e_w, 0, _NUM_EXPERT - 1)
        firsts = (warr == starts[b_of]).astype(jnp.int32)
        seg_lo = jnp.where(valid, off[e_load] - c * _CROWS, 0)
        seg_hi = jnp.where(valid, off_end[e_load] - c * _CROWS, 0)
        metas.append(jnp.stack([b_of, e_load, firsts, seg_lo, seg_hi]
                               ).astype(jnp.int32))
    return dest, metas


def kernel(inp, gate, weight):
    dest, metas = _routing(gate)
    sort_idx = _sc_inverse(dest)
    ys = []
    for c in range(_NCHUNK):
        x_c = _sc_gather_chunk(inp, sort_idx, c)
        ys.append(_grouped_matmul(x_c, weight, metas[c]))
    return _sc_scatter_rows(ys[0], ys[1], sort_idx)


# speedup vs baseline: 1.6344x; 1.6344x over previous
import functools

import jax
import jax.numpy as jnp
from jax import lax
from jax.experimental import pallas as pl
from jax.experimental.pallas import tpu as pltpu
from jax.experimental.pallas import tpu_sc as plsc

_NUM_EXPERT = 8
_IN = 1024
_OUT = 1024
_TOKENS = 2048
_BT = 512
_NB = _TOKENS // _BT
_NW = _NB + _NUM_EXPERT - 1


def _sc_permute(src, idx, scatter):
    rows, feat = src.shape
    mesh = plsc.VectorSubcoreMesh(core_axis_name="c", subcore_axis_name="s")
    nworker = mesh.num_cores * mesh.num_subcores
    per_w = rows // nworker

    @functools.partial(
        pl.kernel,
        mesh=mesh,
        out_type=jax.ShapeDtypeStruct((rows, feat), src.dtype),
        scratch_types=[
            pltpu.VMEM((per_w,), jnp.int32),
            pltpu.VMEM((per_w, feat), src.dtype),
            pltpu.SemaphoreType.DMA,
        ],
    )
    def k(src_hbm, idx_hbm, out_hbm, idx_v, rows_v, sem):
        wid = lax.axis_index("s") * mesh.num_cores + lax.axis_index("c")
        base = wid * per_w
        pltpu.sync_copy(idx_hbm.at[pl.ds(base, per_w)], idx_v)
        if scatter:
            pltpu.sync_copy(src_hbm.at[pl.ds(base, per_w)], rows_v)
            pltpu.async_copy(rows_v, out_hbm.at[idx_v], sem).wait()
        else:
            pltpu.async_copy(src_hbm.at[idx_v], rows_v, sem).wait()
            pltpu.sync_copy(rows_v, out_hbm.at[pl.ds(base, per_w)])

    return k(src, idx)


def _mm_body(meta_ref, x_ref, w_ref, o_ref):
    w = pl.program_id(0)
    first = meta_ref[2, w]
    lo = meta_ref[3, w]
    hi = meta_ref[4, w]
    rows = meta_ref[0, w] * _BT + lax.broadcasted_iota(
        jnp.int32, (_BT, 1), 0)
    mask = (rows >= lo) & (rows < hi)
    xm = jnp.where(mask, x_ref[...], 0.0)
    part = lax.dot_general(
        xm, w_ref[0],
        dimension_numbers=(((1,), (1,)), ((), ())),
        preferred_element_type=jnp.float32,
    )

    @pl.when(first == 1)
    def _():
        o_ref[...] = part

    @pl.when(first == 0)
    def _():
        o_ref[...] += part


def _grouped_matmul(x_sorted, weight, meta):
    grid_spec = pltpu.PrefetchScalarGridSpec(
        num_scalar_prefetch=1,
        grid=(_NW,),
        in_specs=[
            pl.BlockSpec((_BT, _IN), lambda w, m: (m[0, w], 0)),
            pl.BlockSpec((1, _OUT, _IN), lambda w, m: (m[1, w], 0, 0)),
        ],
        out_specs=pl.BlockSpec((_BT, _OUT), lambda w, m: (m[0, w], 0)),
    )
    return pl.pallas_call(
        _mm_body,
        grid_spec=grid_spec,
        out_shape=jax.ShapeDtypeStruct((_TOKENS, _OUT), jnp.float32),
        compiler_params=pltpu.CompilerParams(
            dimension_semantics=("arbitrary",),
        ),
    )(meta, x_sorted, weight)


def _routing(gate):
    g = gate.astype(jnp.int32)
    t = g.shape[0]
    eids = jnp.arange(_NUM_EXPERT, dtype=jnp.int32)
    oh3 = (g.reshape(128, t // 128, 1) == eids).astype(jnp.int32)
    c1 = jnp.cumsum(oh3, axis=1)
    row_tot = c1[:, -1, :]
    c2 = jnp.cumsum(row_tot, axis=0)
    excl = (c1 - oh3) + (c2 - row_tot)[:, None, :]
    pos = jnp.sum(oh3 * excl, axis=2).reshape(t)
    counts = c2[-1]
    off_end = jnp.cumsum(counts)
    off = off_end - counts
    seg = jnp.sum(oh3 * off[None, None, :], axis=2).reshape(t)
    dest = (pos + seg).astype(jnp.int32)
    blk_lo = jnp.arange(_NB, dtype=jnp.int32) * _BT
    e_lo = jnp.sum((off[None, :] <= blk_lo[:, None]).astype(jnp.int32),
                   axis=1) - 1
    e_hi = jnp.sum((off[None, :] <= blk_lo[:, None] + (_BT - 1)
                    ).astype(jnp.int32), axis=1) - 1
    nitem = e_hi - e_lo + 1
    starts = jnp.cumsum(nitem) - nitem
    total = jnp.sum(nitem)
    warr = jnp.arange(_NW, dtype=jnp.int32)
    b_of = jnp.sum((warr[:, None] >= starts[None, :]).astype(jnp.int32),
                   axis=1) - 1
    e_w = e_lo[b_of] + warr - starts[b_of]
    valid = warr < total
    e_load = jnp.clip(e_w, 0, _NUM_EXPERT - 1)
    firsts = (warr == starts[b_of]).astype(jnp.int32)
    seg_lo = jnp.where(valid, off[e_load], 0)
    seg_hi = jnp.where(valid, off_end[e_load], 0)
    meta = jnp.stack([b_of, e_load, firsts, seg_lo, seg_hi]
                     ).astype(jnp.int32)
    return dest, meta


def kernel(inp, gate, weight):
    dest, meta = _routing(gate)
    x_sorted = _sc_permute(inp, dest, scatter=True)
    y_sorted = _grouped_matmul(x_sorted, weight, meta)
    return _sc_permute(y_sorted, dest, scatter=False)


# speedup vs baseline: 1.6366x; 1.0014x over previous
import functools

import jax
import jax.numpy as jnp
from jax import lax
from jax.experimental import pallas as pl
from jax.experimental.pallas import tpu as pltpu
from jax.experimental.pallas import tpu_sc as plsc

_NUM_EXPERT = 8
_IN = 1024
_OUT = 1024
_TOKENS = 2048
_BT = 512
_NB = _TOKENS // _BT
_NW = _NB + _NUM_EXPERT - 1


def _sc_permute(src, idx, scatter):
    rows, feat = src.shape
    mesh = plsc.VectorSubcoreMesh(core_axis_name="c", subcore_axis_name="s")
    nworker = mesh.num_cores * mesh.num_subcores
    per_w = rows // nworker

    @functools.partial(
        pl.kernel,
        mesh=mesh,
        compiler_params=pltpu.CompilerParams(skip_device_barrier=True),
        out_type=jax.ShapeDtypeStruct((rows, feat), src.dtype),
        scratch_types=[
            pltpu.VMEM((per_w,), jnp.int32),
            pltpu.VMEM((per_w, feat), src.dtype),
            pltpu.SemaphoreType.DMA,
        ],
    )
    def k(src_hbm, idx_hbm, out_hbm, idx_v, rows_v, sem):
        wid = lax.axis_index("s") * mesh.num_cores + lax.axis_index("c")
        base = wid * per_w
        pltpu.sync_copy(idx_hbm.at[pl.ds(base, per_w)], idx_v)
        if scatter:
            pltpu.sync_copy(src_hbm.at[pl.ds(base, per_w)], rows_v)
            pltpu.async_copy(rows_v, out_hbm.at[idx_v], sem).wait()
        else:
            pltpu.async_copy(src_hbm.at[idx_v], rows_v, sem).wait()
            pltpu.sync_copy(rows_v, out_hbm.at[pl.ds(base, per_w)])

    return k(src, idx)


def _mm_body(meta_ref, x_ref, w_ref, o_ref):
    w = pl.program_id(0)
    first = meta_ref[2, w]
    lo = meta_ref[3, w]
    hi = meta_ref[4, w]
    rows = meta_ref[0, w] * _BT + lax.broadcasted_iota(
        jnp.int32, (_BT, 1), 0)
    mask = (rows >= lo) & (rows < hi)
    xm = jnp.where(mask, x_ref[...], 0.0)
    part = lax.dot_general(
        xm, w_ref[0],
        dimension_numbers=(((1,), (1,)), ((), ())),
        preferred_element_type=jnp.float32,
    )

    @pl.when(first == 1)
    def _():
        o_ref[...] = part

    @pl.when(first == 0)
    def _():
        o_ref[...] += part


def _grouped_matmul(x_sorted, weight, meta):
    grid_spec = pltpu.PrefetchScalarGridSpec(
        num_scalar_prefetch=1,
        grid=(_NW,),
        in_specs=[
            pl.BlockSpec((_BT, _IN), lambda w, m: (m[0, w], 0)),
            pl.BlockSpec((1, _OUT, _IN), lambda w, m: (m[1, w], 0, 0)),
        ],
        out_specs=pl.BlockSpec((_BT, _OUT), lambda w, m: (m[0, w], 0)),
    )
    return pl.pallas_call(
        _mm_body,
        grid_spec=grid_spec,
        out_shape=jax.ShapeDtypeStruct((_TOKENS, _OUT), jnp.float32),
        compiler_params=pltpu.CompilerParams(
            dimension_semantics=("arbitrary",),
            skip_device_barrier=True,
        ),
    )(meta, x_sorted, weight)


def _routing(gate):
    g = gate.astype(jnp.int32)
    t = g.shape[0]
    eids = jnp.arange(_NUM_EXPERT, dtype=jnp.int32)
    oh3 = (g.reshape(128, t // 128, 1) == eids).astype(jnp.int32)
    c1 = jnp.cumsum(oh3, axis=1)
    row_tot = c1[:, -1, :]
    c2 = jnp.cumsum(row_tot, axis=0)
    excl = (c1 - oh3) + (c2 - row_tot)[:, None, :]
    pos = jnp.sum(oh3 * excl, axis=2).reshape(t)
    counts = c2[-1]
    off_end = jnp.cumsum(counts)
    off = off_end - counts
    seg = jnp.sum(oh3 * off[None, None, :], axis=2).reshape(t)
    dest = (pos + seg).astype(jnp.int32)
    blk_lo = jnp.arange(_NB, dtype=jnp.int32) * _BT
    e_lo = jnp.sum((off[None, :] <= blk_lo[:, None]).astype(jnp.int32),
                   axis=1) - 1
    e_hi = jnp.sum((off[None, :] <= blk_lo[:, None] + (_BT - 1)
                    ).astype(jnp.int32), axis=1) - 1
    nitem = e_hi - e_lo + 1
    starts = jnp.cumsum(nitem) - nitem
    total = jnp.sum(nitem)
    warr = jnp.arange(_NW, dtype=jnp.int32)
    b_of = jnp.sum((warr[:, None] >= starts[None, :]).astype(jnp.int32),
                   axis=1) - 1
    e_w = e_lo[b_of] + warr - starts[b_of]
    valid = warr < total
    e_load = jnp.clip(e_w, 0, _NUM_EXPERT - 1)
    firsts = (warr == starts[b_of]).astype(jnp.int32)
    seg_lo = jnp.where(valid, off[e_load], 0)
    seg_hi = jnp.where(valid, off_end[e_load], 0)
    meta = jnp.stack([b_of, e_load, firsts, seg_lo, seg_hi]
                     ).astype(jnp.int32)
    return dest, meta


def kernel(inp, gate, weight):
    dest, meta = _routing(gate)
    x_sorted = _sc_permute(inp, dest, scatter=True)
    y_sorted = _grouped_matmul(x_sorted, weight, meta)
    return _sc_permute(y_sorted, dest, scatter=False)


# speedup vs baseline: 1.6380x; 1.0008x over previous
import functools

import jax
import jax.numpy as jnp
from jax import lax
from jax.experimental import pallas as pl
from jax.experimental.pallas import tpu as pltpu
from jax.experimental.pallas import tpu_sc as plsc

_NUM_EXPERT = 8
_IN = 1024
_OUT = 1024
_TOKENS = 2048
_BT = 512
_NB = _TOKENS // _BT
_NW = _NB + _NUM_EXPERT - 1


def _sc_permute(src, idx, scatter):
    rows, feat = src.shape
    mesh = plsc.VectorSubcoreMesh(core_axis_name="c", subcore_axis_name="s")
    nworker = mesh.num_cores * mesh.num_subcores
    per_w = rows // nworker

    @functools.partial(
        pl.kernel,
        mesh=mesh,
        out_type=jax.ShapeDtypeStruct((rows, feat), src.dtype),
        scratch_types=[
            pltpu.VMEM((per_w,), jnp.int32),
            pltpu.VMEM((per_w, feat), src.dtype),
            pltpu.SemaphoreType.DMA,
        ],
    )
    def k(src_hbm, idx_hbm, out_hbm, idx_v, rows_v, sem):
        wid = lax.axis_index("s") * mesh.num_cores + lax.axis_index("c")
        base = wid * per_w
        pltpu.sync_copy(idx_hbm.at[pl.ds(base, per_w)], idx_v)
        if scatter:
            pltpu.sync_copy(src_hbm.at[pl.ds(base, per_w)], rows_v)
            pltpu.async_copy(rows_v, out_hbm.at[idx_v], sem).wait()
        else:
            pltpu.async_copy(src_hbm.at[idx_v], rows_v, sem).wait()
            pltpu.sync_copy(rows_v, out_hbm.at[pl.ds(base, per_w)])

    return k(src, idx)


def _mm_body(meta_ref, x_ref, w_ref, o_ref):
    w = pl.program_id(0)
    first = meta_ref[2, w]
    lo = meta_ref[3, w]
    hi = meta_ref[4, w]
    rows = meta_ref[0, w] * _BT + lax.broadcasted_iota(
        jnp.int32, (_BT, 1), 0)
    mask = (rows >= lo) & (rows < hi)
    xm = jnp.where(mask, x_ref[...], 0.0)
    part = lax.dot_general(
        xm, w_ref[0],
        dimension_numbers=(((1,), (1,)), ((), ())),
        preferred_element_type=jnp.float32,
    )

    @pl.when(first == 1)
    def _():
        o_ref[...] = part

    @pl.when(first == 0)
    def _():
        o_ref[...] += part


def _grouped_matmul(x_sorted, weight, meta):
    grid_spec = pltpu.PrefetchScalarGridSpec(
        num_scalar_prefetch=1,
        grid=(_NW,),
        in_specs=[
            pl.BlockSpec((_BT, _IN), lambda w, m: (m[0, w], 0)),
            pl.BlockSpec((1, _OUT, _IN), lambda w, m: (m[1, w], 0, 0)),
        ],
        out_specs=pl.BlockSpec((_BT, _OUT), lambda w, m: (m[0, w], 0)),
    )
    return pl.pallas_call(
        _mm_body,
        grid_spec=grid_spec,
        out_shape=jax.ShapeDtypeStruct((_TOKENS, _OUT), jnp.float32),
        compiler_params=pltpu.CompilerParams(
            dimension_semantics=("arbitrary",),
        ),
    )(meta, x_sorted, weight)


def _routing(gate):
    g = gate.astype(jnp.int32)
    t = g.shape[0]
    eids = jnp.arange(_NUM_EXPERT, dtype=jnp.int32)
    oh3 = (g.reshape(128, t // 128, 1) == eids).astype(jnp.int32)
    c1 = jnp.cumsum(oh3, axis=1)
    row_tot = c1[:, -1, :]
    c2 = jnp.cumsum(row_tot, axis=0)
    excl = (c1 - oh3) + (c2 - row_tot)[:, None, :]
    pos = jnp.sum(oh3 * excl, axis=2).reshape(t)
    counts = c2[-1]
    off_end = jnp.cumsum(counts)
    off = off_end - counts
    seg = jnp.sum(oh3 * off[None, None, :], axis=2).reshape(t)
    dest = (pos + seg).astype(jnp.int32)
    blk_lo = jnp.arange(_NB, dtype=jnp.int32) * _BT
    e_lo = jnp.sum((off[None, :] <= blk_lo[:, None]).astype(jnp.int32),
                   axis=1) - 1
    e_hi = jnp.sum((off[None, :] <= blk_lo[:, None] + (_BT - 1)
                    ).astype(jnp.int32), axis=1) - 1
    nitem = e_hi - e_lo + 1
    starts = jnp.cumsum(nitem) - nitem
    total = jnp.sum(nitem)
    warr = jnp.arange(_NW, dtype=jnp.int32)
    b_of = jnp.sum((warr[:, None] >= starts[None, :]).astype(jnp.int32),
                   axis=1) - 1
    e_w = e_lo[b_of] + warr - starts[b_of]
    valid = warr < total
    e_load = jnp.clip(e_w, 0, _NUM_EXPERT - 1)
    firsts = (warr == starts[b_of]).astype(jnp.int32)
    seg_lo = jnp.where(valid, off[e_load], 0)
    seg_hi = jnp.where(valid, off_end[e_load], 0)
    meta = jnp.stack([b_of, e_load, firsts, seg_lo, seg_hi]
                     ).astype(jnp.int32)
    return dest, meta


def kernel(inp, gate, weight):
    dest, meta = _routing(gate)
    x_sorted = _sc_permute(inp, dest, scatter=True)
    y_sorted = _grouped_matmul(x_sorted, weight, meta)
    return _sc_permute(y_sorted, dest, scatter=False)
